# baseline (speedup 1.0000x reference)
import jax
import jax.numpy as jnp
from jax.experimental import pallas as pl
from jax.experimental.pallas import tpu as pltpu

_CH = 256


def _body(idx_ref, table_ref, out_ref, scratch, sem):
    r = idx_ref[0] % 8
    scratch[...] = jnp.broadcast_to(table_ref[pl.ds(r, 1), :], scratch.shape)
    n = out_ref.shape[0] // _CH
    copies = [
        pltpu.make_async_copy(scratch, out_ref.at[pl.ds(k * _CH, _CH), :], sem)
        for k in range(n)
    ]
    for c in copies:
        c.start()
    for c in copies:
        c.wait()


def kernel(inputs, num_particle, lookup_table):
    B, T, D = inputs.shape
    rows = B * T
    idx = jnp.asarray(num_particle, jnp.int32).reshape(1)
    out = pl.pallas_call(
        _body,
        grid_spec=pltpu.PrefetchScalarGridSpec(
            num_scalar_prefetch=1,
            grid=(1,),
            in_specs=[pl.BlockSpec((8, D), lambda i, idx_ref: (idx_ref[0] // 8, 0))],
            out_specs=pl.BlockSpec(memory_space=pltpu.MemorySpace.HBM),
            scratch_shapes=[
                pltpu.VMEM((_CH, D), jnp.float32),
                pltpu.SemaphoreType.DMA,
            ],
        ),
        out_shape=jax.ShapeDtypeStruct((rows, D), jnp.float32),
    )(idx, lookup_table)
    return out.reshape(B, T, D)


# speedup vs baseline: 1.0062x; 1.0062x over previous
import jax
import jax.numpy as jnp
from jax.experimental import pallas as pl
from jax.experimental.pallas import tpu as pltpu

_CH = 128


def _body(idx_ref, table_ref, out_ref, scratch, sem):
    r = idx_ref[0] % 8
    scratch[...] = jnp.broadcast_to(table_ref[pl.ds(r, 1), :], scratch.shape)
    n = out_ref.shape[0] // _CH
    copies = [
        pltpu.make_async_copy(scratch, out_ref.at[pl.ds(k * _CH, _CH), :], sem)
        for k in range(n)
    ]
    for c in copies:
        c.start()
    for c in copies:
        c.wait()


def kernel(inputs, num_particle, lookup_table):
    B, T, D = inputs.shape
    rows = B * T
    idx = jnp.asarray(num_particle, jnp.int32).reshape(1)
    out = pl.pallas_call(
        _body,
        grid_spec=pltpu.PrefetchScalarGridSpec(
            num_scalar_prefetch=1,
            grid=(1,),
            in_specs=[pl.BlockSpec((8, D), lambda i, idx_ref: (idx_ref[0] // 8, 0))],
            out_specs=pl.BlockSpec(memory_space=pltpu.MemorySpace.HBM),
            scratch_shapes=[
                pltpu.VMEM((_CH, D), jnp.float32),
                pltpu.SemaphoreType.DMA,
            ],
        ),
        out_shape=jax.ShapeDtypeStruct((rows, D), jnp.float32),
    )(idx, lookup_table)
    return out.reshape(B, T, D)
